# baseline (speedup 1.0000x reference)
import functools

import jax
import jax.numpy as jnp
from jax import lax
from jax.experimental import pallas as pl
from jax.experimental.pallas import tpu as pltpu
from jax.experimental.pallas import tpu_sc as plsc

_CHUNK = 64
_NCORES = 2
_NSUB = 16
_NWORKERS = _NCORES * _NSUB


def _sc_degree(dst2d, iota, zeros, *, nch_w, npad):
    mesh = plsc.VectorSubcoreMesh(core_axis_name="c", subcore_axis_name="s")
    hrows = npad // 128

    @functools.partial(
        pl.kernel,
        out_type=jax.ShapeDtypeStruct((_NCORES, hrows, 128), jnp.float32),
        mesh=mesh,
        compiler_params=pltpu.CompilerParams(needs_layout_passes=False),
        scratch_types=[
            pltpu.VMEM((nch_w, _CHUNK), jnp.int32),
            pltpu.VMEM((hrows, 128), jnp.float32),
            pltpu.VMEM((hrows,), jnp.int32),
            pltpu.VMEM_SHARED((hrows, 128), jnp.float32),
        ],
    )
    def k(dst_hbm, iota_hbm, zeros_hbm, out_hbm, dst_v, hist, iota_v, acc):
        c = lax.axis_index("c")
        s = lax.axis_index("s")
        wid = s * _NCORES + c
        pltpu.sync_copy(dst_hbm.at[pl.ds(wid * nch_w, nch_w)], dst_v)
        pltpu.sync_copy(iota_hbm, iota_v)
        pltpu.sync_copy(zeros_hbm, hist)

        @pl.when(s == 0)
        def _():
            pltpu.sync_copy(zeros_hbm, acc)

        plsc.subcore_barrier()

        ones = jnp.ones((16,), jnp.float32)

        vb = _CHUNK // 16

        def body(i, carry):
            idx = dst_v[i // vb, pl.ds((i % vb) * 16, 16)]
            plsc.addupdate_scatter(hist, [idx >> 7, idx & 127], ones)
            return carry

        lax.fori_loop(0, nch_w * vb, body, 0)
        pltpu.sync_copy(hist, acc.at[iota_v], add=True)
        plsc.subcore_barrier()

        @pl.when(s == 0)
        def _():
            pltpu.sync_copy(acc, out_hbm.at[c])

    return k(dst2d, iota, zeros)


def _sc_scatter(gl, gr, sd, zeros, *, nch_w, npad, rows_s, fh):
    mesh = plsc.VectorSubcoreMesh(core_axis_name="c", subcore_axis_name="s")

    @functools.partial(
        pl.kernel,
        out_type=jax.ShapeDtypeStruct((_NCORES, 2, npad, fh), jnp.float32),
        mesh=mesh,
        compiler_params=pltpu.CompilerParams(use_tc_tiling_on_sc=False),
        scratch_types=[
            pltpu.VMEM((2, nch_w, _CHUNK), jnp.int32),
            pltpu.VMEM((_CHUNK, fh), jnp.float32),
            pltpu.VMEM((_CHUNK, fh), jnp.float32),
            pltpu.VMEM_SHARED((npad, fh), jnp.float32),
            pltpu.SemaphoreType.DMA,
            pltpu.SemaphoreType.DMA,
            pltpu.SemaphoreType.DMA,
            pltpu.SemaphoreType.DMA,
        ],
    )
    def k(gl_hbm, gr_hbm, sd_hbm, zeros_hbm, out_hbm, sd_v, rows_a, rows_b,
          acc, sem_a, sem_b, sem_sa, sem_sb):
        c = lax.axis_index("c")
        s = lax.axis_index("s")
        wid = s * _NCORES + c
        pltpu.sync_copy(sd_hbm.at[wid], sd_v)
        pltpu.sync_copy(zeros_hbm.at[pl.ds(s * rows_s, rows_s)],
                        acc.at[pl.ds(s * rows_s, rows_s)])
        plsc.subcore_barrier()

        for h, g_hbm in enumerate((gl_hbm, gr_hbm)):
            pltpu.async_copy(g_hbm.at[sd_v.at[0, 0]], rows_a, sem_a)

            def body(j, carry):
                i = 2 * j
                pltpu.make_async_copy(g_hbm.at[sd_v.at[0, i]], rows_a,
                                      sem_a).wait()
                pltpu.async_copy(rows_a, acc.at[sd_v.at[1, i]], sem_sa,
                                 add=True)

                @pl.when(j > 0)
                def _():
                    pltpu.make_async_copy(rows_b, acc.at[sd_v.at[1, i]],
                                          sem_sb).wait()

                pltpu.async_copy(g_hbm.at[sd_v.at[0, i + 1]], rows_b, sem_b)
                pltpu.make_async_copy(g_hbm.at[sd_v.at[0, i + 1]], rows_b,
                                      sem_b).wait()
                pltpu.async_copy(rows_b, acc.at[sd_v.at[1, i + 1]], sem_sb,
                                 add=True)
                pltpu.make_async_copy(rows_a, acc.at[sd_v.at[1, i]],
                                      sem_sa).wait()

                @pl.when(i + 2 < nch_w)
                def _():
                    pltpu.async_copy(g_hbm.at[sd_v.at[0, i + 2]], rows_a,
                                     sem_a)

                return carry

            lax.fori_loop(0, nch_w // 2, body, 0)
            pltpu.make_async_copy(rows_b, acc.at[sd_v.at[1, 0]],
                                  sem_sb).wait()
            plsc.subcore_barrier()
            pltpu.sync_copy(acc.at[pl.ds(s * rows_s, rows_s)],
                            out_hbm.at[c, h, pl.ds(s * rows_s, rows_s)])
            if h == 0:
                pltpu.sync_copy(zeros_hbm.at[pl.ds(s * rows_s, rows_s)],
                                acc.at[pl.ds(s * rows_s, rows_s)])
                plsc.subcore_barrier()

    return k(gl, gr, sd, zeros)


def _dinv(d0_ref, d1_ref):
    deg = d0_ref[...] + d1_ref[...] + 1.0
    return lax.rsqrt(jnp.maximum(deg, 1.0))


def _agg(parts, gl_ref, gr_ref):
    left = parts[0][0][...] + parts[1][0][...] + gl_ref[...]
    right = parts[0][1][...] + parts[1][1][...] + gr_ref[...]
    return jnp.concatenate([left, right], axis=1)


def _tc_first(xpad, w1, d0, d1, *, npad, hid):
    def body(x_ref, w_ref, d0_ref, d1_ref, gl_ref, gr_ref):
        h = jnp.dot(x_ref[...], w_ref[...], preferred_element_type=jnp.float32)
        g = _dinv(d0_ref, d1_ref) * h
        gl_ref[...] = g[:, :hid // 2]
        gr_ref[...] = g[:, hid // 2:]

    return pl.pallas_call(
        body,
        out_shape=[jax.ShapeDtypeStruct((npad, hid // 2), jnp.float32),
                   jax.ShapeDtypeStruct((npad, hid // 2), jnp.float32)],
    )(xpad, w1, d0, d1)


def _tc_mid(p, g1l, g1r, d0, d1, w2, b1, *, npad, hid):
    def body(p00, p01, p10, p11, g1l_ref, g1r_ref, d0_ref, d1_ref, w_ref,
             b_ref, gl_ref, gr_ref):
        dinv = _dinv(d0_ref, d1_ref)
        agg = _agg(((p00, p01), (p10, p11)), g1l_ref, g1r_ref)
        h = jnp.maximum(dinv * agg + b_ref[...], 0.0)
        g = dinv * jnp.dot(h, w_ref[...], preferred_element_type=jnp.float32)
        gl_ref[...] = g[:, :hid // 2]
        gr_ref[...] = g[:, hid // 2:]

    return pl.pallas_call(
        body,
        out_shape=[jax.ShapeDtypeStruct((npad, hid // 2), jnp.float32),
                   jax.ShapeDtypeStruct((npad, hid // 2), jnp.float32)],
    )(p[0, 0], p[0, 1], p[1, 0], p[1, 1], g1l, g1r, d0, d1, w2, b1)


def _tc_last(p, g2l, g2r, d0, d1, wh, b2, bh, *, npad, hid, ncls):
    def body(p00, p01, p10, p11, g2l_ref, g2r_ref, d0_ref, d1_ref, w_ref,
             b2_ref, bh_ref, s_ref, h_ref):
        dinv = _dinv(d0_ref, d1_ref)
        agg = _agg(((p00, p01), (p10, p11)), g2l_ref, g2r_ref)
        h = jnp.maximum(dinv * agg + b2_ref[...], 0.0)
        h_ref[...] = h
        s_ref[...] = jnp.dot(h, w_ref[...],
                             preferred_element_type=jnp.float32) + bh_ref[...]

    return pl.pallas_call(
        body,
        out_shape=[jax.ShapeDtypeStruct((npad, ncls), jnp.float32),
                   jax.ShapeDtypeStruct((npad, hid), jnp.float32)],
    )(p[0, 0], p[0, 1], p[1, 0], p[1, 1], g2l, g2r, d0, d1, wh, b2, bh)


def kernel(x, edge_index, W1, b1, W2, b2, Wh, bh):
    n, f_in = x.shape
    hid = W1.shape[1]
    ncls = Wh.shape[1]
    e = edge_index.shape[1]

    npad = ((n + 1 + 127) // 128) * 128
    rows_s = npad // _NSUB
    nch_w = -(-e // (_NWORKERS * _CHUNK))
    nch_w = ((nch_w + 7) // 8) * 8
    epad = _NWORKERS * nch_w * _CHUNK

    pad = jnp.full((epad - e,), n, dtype=edge_index.dtype)
    src2d = jnp.concatenate([edge_index[0], pad]).reshape(-1, _CHUNK)
    dst2d = jnp.concatenate([edge_index[1], pad]).reshape(-1, _CHUNK)
    sd = jnp.stack([src2d.reshape(_NWORKERS, nch_w, _CHUNK),
                    dst2d.reshape(_NWORKERS, nch_w, _CHUNK)], axis=1)
    xpad = jnp.pad(x, ((0, npad - n), (0, 0)))
    fh = hid // 2
    zeros = jnp.zeros((npad, fh), jnp.float32)
    zeros_deg = jnp.zeros((npad // 128, 128), jnp.float32)
    iota = jnp.arange(npad // 128, dtype=jnp.int32)

    degp = _sc_degree(dst2d, iota, zeros_deg, nch_w=nch_w, npad=npad)
    degp = degp.reshape(_NCORES, npad, 1)
    d0, d1 = degp[0], degp[1]

    g1l, g1r = _tc_first(xpad, W1, d0, d1, npad=npad, hid=hid)
    parts1 = _sc_scatter(g1l, g1r, sd, zeros,
                         nch_w=nch_w, npad=npad, rows_s=rows_s, fh=fh)
    g2l, g2r = _tc_mid(parts1, g1l, g1r, d0, d1, W2, b1, npad=npad, hid=hid)
    parts2 = _sc_scatter(g2l, g2r, sd, zeros,
                         nch_w=nch_w, npad=npad, rows_s=rows_s, fh=fh)
    scores, h = _tc_last(parts2, g2l, g2r, d0, d1, Wh, b2, bh,
                         npad=npad, hid=hid, ncls=ncls)
    return (scores[:n], h[:n])


# speedup vs baseline: 2.0799x; 2.0799x over previous
import functools

import jax
import jax.numpy as jnp
from jax import lax
from jax.experimental import pallas as pl
from jax.experimental.pallas import tpu as pltpu
from jax.experimental.pallas import tpu_sc as plsc

_CHUNK = 64
_NCORES = 2
_NSUB = 16
_NWORKERS = _NCORES * _NSUB


def _sc_degree(dst2d, iota, zeros, *, nch_w, npad):
    mesh = plsc.VectorSubcoreMesh(core_axis_name="c", subcore_axis_name="s")
    hrows = npad // 128

    @functools.partial(
        pl.kernel,
        out_type=jax.ShapeDtypeStruct((_NCORES, hrows, 128), jnp.float32),
        mesh=mesh,
        compiler_params=pltpu.CompilerParams(needs_layout_passes=False),
        scratch_types=[
            pltpu.VMEM((nch_w, _CHUNK), jnp.int32),
            pltpu.VMEM((hrows, 128), jnp.float32),
            pltpu.VMEM((hrows,), jnp.int32),
            pltpu.VMEM_SHARED((hrows, 128), jnp.float32),
        ],
    )
    def k(dst_hbm, iota_hbm, zeros_hbm, out_hbm, dst_v, hist, iota_v, acc):
        c = lax.axis_index("c")
        s = lax.axis_index("s")
        wid = s * _NCORES + c
        pltpu.sync_copy(dst_hbm.at[pl.ds(wid * nch_w, nch_w)], dst_v)
        pltpu.sync_copy(iota_hbm, iota_v)
        pltpu.sync_copy(zeros_hbm, hist)

        @pl.when(s == 0)
        def _():
            pltpu.sync_copy(zeros_hbm, acc)

        plsc.subcore_barrier()

        ones = jnp.ones((16,), jnp.float32)

        vb = _CHUNK // 16

        def body(i, carry):
            idx = dst_v[i // vb, pl.ds((i % vb) * 16, 16)]
            plsc.addupdate_scatter(hist, [idx >> 7, idx & 127], ones)
            return carry

        lax.fori_loop(0, nch_w * vb, body, 0)
        pltpu.sync_copy(hist, acc.at[iota_v], add=True)
        plsc.subcore_barrier()

        @pl.when(s == 0)
        def _():
            pltpu.sync_copy(acc, out_hbm.at[c])

    return k(dst2d, iota, zeros)


def _sc_scatter(gl, gr, sd, zeros, *, nch_w, npad, rows_s, fh):
    mesh = plsc.VectorSubcoreMesh(core_axis_name="c", subcore_axis_name="s")

    @functools.partial(
        pl.kernel,
        out_type=jax.ShapeDtypeStruct((_NCORES, 2, npad, fh), jnp.float32),
        mesh=mesh,
        compiler_params=pltpu.CompilerParams(use_tc_tiling_on_sc=False),
        scratch_types=[
            pltpu.VMEM((2, nch_w, _CHUNK), jnp.int32),
            pltpu.VMEM((_CHUNK, fh), jnp.float32),
            pltpu.VMEM_SHARED((npad, fh), jnp.float32),
            pltpu.VMEM_SHARED((npad, fh), jnp.float32),
            pltpu.SemaphoreType.DMA,
        ],
    )
    def k(gl_hbm, gr_hbm, sd_hbm, zeros_hbm, out_hbm, sd_v, rows_a,
          acc, gtab, sem_a):
        c = lax.axis_index("c")
        s = lax.axis_index("s")
        wid = s * _NCORES + c
        sl = pl.ds(s * rows_s, rows_s)
        pltpu.sync_copy(sd_hbm.at[wid], sd_v)
        pltpu.sync_copy(zeros_hbm.at[sl], acc.at[sl])
        pltpu.sync_copy(gl_hbm.at[sl], gtab.at[sl])
        plsc.subcore_barrier()

        for h, g_hbm in enumerate((gl_hbm, gr_hbm)):
            def body(i, carry):
                pltpu.async_copy(gtab.at[sd_v.at[0, i]], rows_a,
                                 sem_a).wait()
                pltpu.sync_copy(rows_a, acc.at[sd_v.at[1, i]], add=True)
                return carry

            lax.fori_loop(0, nch_w, body, 0)
            plsc.subcore_barrier()
            pltpu.sync_copy(acc.at[sl], out_hbm.at[c, h, sl])
            if h == 0:
                pltpu.sync_copy(zeros_hbm.at[sl], acc.at[sl])
                pltpu.sync_copy(gr_hbm.at[sl], gtab.at[sl])
                plsc.subcore_barrier()

    return k(gl, gr, sd, zeros)


def _dinv(d0_ref, d1_ref):
    deg = d0_ref[...] + d1_ref[...] + 1.0
    return lax.rsqrt(jnp.maximum(deg, 1.0))


def _agg(parts, gl_ref, gr_ref):
    left = parts[0][0][...] + parts[1][0][...] + gl_ref[...]
    right = parts[0][1][...] + parts[1][1][...] + gr_ref[...]
    return jnp.concatenate([left, right], axis=1)


def _tc_first(xpad, w1, d0, d1, *, npad, hid):
    def body(x_ref, w_ref, d0_ref, d1_ref, gl_ref, gr_ref):
        h = jnp.dot(x_ref[...], w_ref[...], preferred_element_type=jnp.float32)
        g = _dinv(d0_ref, d1_ref) * h
        gl_ref[...] = g[:, :hid // 2]
        gr_ref[...] = g[:, hid // 2:]

    return pl.pallas_call(
        body,
        out_shape=[jax.ShapeDtypeStruct((npad, hid // 2), jnp.float32),
                   jax.ShapeDtypeStruct((npad, hid // 2), jnp.float32)],
    )(xpad, w1, d0, d1)


def _tc_mid(p, g1l, g1r, d0, d1, w2, b1, *, npad, hid):
    def body(p00, p01, p10, p11, g1l_ref, g1r_ref, d0_ref, d1_ref, w_ref,
             b_ref, gl_ref, gr_ref):
        dinv = _dinv(d0_ref, d1_ref)
        agg = _agg(((p00, p01), (p10, p11)), g1l_ref, g1r_ref)
        h = jnp.maximum(dinv * agg + b_ref[...], 0.0)
        g = dinv * jnp.dot(h, w_ref[...], preferred_element_type=jnp.float32)
        gl_ref[...] = g[:, :hid // 2]
        gr_ref[...] = g[:, hid // 2:]

    return pl.pallas_call(
        body,
        out_shape=[jax.ShapeDtypeStruct((npad, hid // 2), jnp.float32),
                   jax.ShapeDtypeStruct((npad, hid // 2), jnp.float32)],
    )(p[0, 0], p[0, 1], p[1, 0], p[1, 1], g1l, g1r, d0, d1, w2, b1)


def _tc_last(p, g2l, g2r, d0, d1, wh, b2, bh, *, npad, hid, ncls):
    def body(p00, p01, p10, p11, g2l_ref, g2r_ref, d0_ref, d1_ref, w_ref,
             b2_ref, bh_ref, s_ref, h_ref):
        dinv = _dinv(d0_ref, d1_ref)
        agg = _agg(((p00, p01), (p10, p11)), g2l_ref, g2r_ref)
        h = jnp.maximum(dinv * agg + b2_ref[...], 0.0)
        h_ref[...] = h
        s_ref[...] = jnp.dot(h, w_ref[...],
                             preferred_element_type=jnp.float32) + bh_ref[...]

    return pl.pallas_call(
        body,
        out_shape=[jax.ShapeDtypeStruct((npad, ncls), jnp.float32),
                   jax.ShapeDtypeStruct((npad, hid), jnp.float32)],
    )(p[0, 0], p[0, 1], p[1, 0], p[1, 1], g2l, g2r, d0, d1, wh, b2, bh)


def kernel(x, edge_index, W1, b1, W2, b2, Wh, bh):
    n, f_in = x.shape
    hid = W1.shape[1]
    ncls = Wh.shape[1]
    e = edge_index.shape[1]

    npad = ((n + 1 + 127) // 128) * 128
    rows_s = npad // _NSUB
    nch_w = -(-e // (_NWORKERS * _CHUNK))
    nch_w = ((nch_w + 7) // 8) * 8
    epad = _NWORKERS * nch_w * _CHUNK

    pad = jnp.full((epad - e,), n, dtype=edge_index.dtype)
    src2d = jnp.concatenate([edge_index[0], pad]).reshape(-1, _CHUNK)
    dst2d = jnp.concatenate([edge_index[1], pad]).reshape(-1, _CHUNK)
    sd = jnp.stack([src2d.reshape(_NWORKERS, nch_w, _CHUNK),
                    dst2d.reshape(_NWORKERS, nch_w, _CHUNK)], axis=1)
    xpad = jnp.pad(x, ((0, npad - n), (0, 0)))
    fh = hid // 2
    zeros = jnp.zeros((npad, fh), jnp.float32)
    zeros_deg = jnp.zeros((npad // 128, 128), jnp.float32)
    iota = jnp.arange(npad // 128, dtype=jnp.int32)

    degp = _sc_degree(dst2d, iota, zeros_deg, nch_w=nch_w, npad=npad)
    degp = degp.reshape(_NCORES, npad, 1)
    d0, d1 = degp[0], degp[1]

    g1l, g1r = _tc_first(xpad, W1, d0, d1, npad=npad, hid=hid)
    parts1 = _sc_scatter(g1l, g1r, sd, zeros,
                         nch_w=nch_w, npad=npad, rows_s=rows_s, fh=fh)
    g2l, g2r = _tc_mid(parts1, g1l, g1r, d0, d1, W2, b1, npad=npad, hid=hid)
    parts2 = _sc_scatter(g2l, g2r, sd, zeros,
                         nch_w=nch_w, npad=npad, rows_s=rows_s, fh=fh)
    scores, h = _tc_last(parts2, g2l, g2r, d0, d1, Wh, b2, bh,
                         npad=npad, hid=hid, ncls=ncls)
    return (scores[:n], h[:n])
